# Initial kernel scaffold; baseline (speedup 1.0000x reference)
import functools

import jax
import jax.numpy as jnp
from jax import lax
from jax.experimental import pallas as pl
from jax.experimental.pallas import tpu as pltpu
from jax.experimental.pallas import tpu_sc as plsc

N_CAND = 16384
POOL_SIZE = 4096
SUMMARY_DIM = 128
THRESHOLD = 0.5
L = 16
NW = 32
ROWS_PER_TILE = POOL_SIZE // NW

_mesh = plsc.VectorSubcoreMesh(core_axis_name="c", subcore_axis_name="s")


def _iota():
    return lax.iota(jnp.int32, L)


@functools.partial(
    pl.kernel,
    out_type=[
        jax.ShapeDtypeStruct((POOL_SIZE,), jnp.float32),
        jax.ShapeDtypeStruct((POOL_SIZE,), jnp.int32),
        jax.ShapeDtypeStruct((L,), jnp.int32),
    ],
    mesh=_mesh,
    scratch_types=[
        pltpu.VMEM((N_CAND,), jnp.float32),
        pltpu.VMEM((N_CAND,), jnp.float32),
        pltpu.VMEM((N_CAND,), jnp.int32),
        pltpu.VMEM((POOL_SIZE,), jnp.float32),
        pltpu.VMEM((POOL_SIZE,), jnp.int32),
        pltpu.VMEM((POOL_SIZE // L,), jnp.float32),
        pltpu.VMEM((L,), jnp.float32),
        pltpu.VMEM((L,), jnp.int32),
    ],
)
def _k1(scores_hbm, priorities_hbm, pri_out, cand_out, stats_out,
        sc_v, vsc_v, vid_v, pri_v, cand_v, g1_v, g2_v, stat_v):
    wid = lax.axis_index("s") * 2 + lax.axis_index("c")

    @pl.when(wid == 0)
    def _():
        pltpu.sync_copy(scores_hbm, sc_v)
        pltpu.sync_copy(priorities_hbm, pri_v)
        iot = _iota()

        def compact_body(c, off_v):
            v = sc_v[pl.ds(c * L, L)]
            msk = v > THRESHOLD
            incl = jnp.cumsum(msk.astype(jnp.int32))
            pos = off_v + incl - 1
            ids = c * L + iot
            plsc.store_scatter(vsc_v, [pos], v, mask=msk)
            plsc.store_scatter(vid_v, [pos], ids, mask=msk)
            return off_v + plsc.all_reduce_population_count(msk)

        off_v = lax.fori_loop(0, N_CAND // L, compact_body,
                              jnp.zeros((L,), jnp.int32))
        nvalid = jnp.max(off_v)
        k_count = jnp.minimum(nvalid, POOL_SIZE)
        m_rep = nvalid - k_count

        def insert_body(ci, _):
            base = ci * L
            msk = (base + iot) < k_count
            scv = vsc_v[pl.ds(base, L)]
            idv = vid_v[pl.ds(base, L)]
            pold = pri_v[pl.ds(base, L)]
            pri_v[pl.ds(base, L)] = jnp.where(msk, scv, pold)
            cand_v[pl.ds(base, L)] = jnp.where(msk, idv, -1)
            return 0

        lax.fori_loop(0, POOL_SIZE // L, insert_body, 0)

        lane0 = iot == 0

        def g1_body(j, _):
            x = pri_v[pl.ds(j * L, L)]
            mn = jnp.min(x)
            plsc.store_scatter(g1_v, [jnp.full((L,), j, jnp.int32)],
                               jnp.full((L,), mn, jnp.float32), mask=lane0)
            return 0

        lax.fori_loop(0, POOL_SIZE // L, g1_body, 0)

        def g2_body(g, _):
            x = g1_v[pl.ds(g * L, L)]
            mn = jnp.min(x)
            plsc.store_scatter(g2_v, [jnp.full((L,), g, jnp.int32)],
                               jnp.full((L,), mn, jnp.float32), mask=lane0)
            return 0

        lax.fori_loop(0, L, g2_body, 0)

        def rep_body(t, _):
            posv = jnp.full((L,), POOL_SIZE + t, jnp.int32)
            rv = plsc.load_gather(vsc_v, [posv])
            g2v = g2_v[...]

            @pl.when(jnp.any(rv > g2v))
            def _():
                gmin = jnp.min(g2v)
                l2 = jnp.min(jnp.where(g2v == gmin, iot, L))
                g1c = g1_v[pl.ds(l2 * L, L)]
                l1 = jnp.min(jnp.where(g1c == gmin, iot, L))
                grp = l2 * L + l1
                pc = pri_v[pl.ds(grp * L, L)]
                l0 = jnp.min(jnp.where(pc == gmin, iot, L))
                newp = jnp.where(iot == l0, rv, pc)
                pri_v[pl.ds(grp * L, L)] = newp
                idv = plsc.load_gather(vid_v, [posv])
                cc = cand_v[pl.ds(grp * L, L)]
                cand_v[pl.ds(grp * L, L)] = jnp.where(iot == l0, idv, cc)
                ng1 = jnp.min(newp)
                g1c2 = jnp.where(iot == l1, ng1, g1c)
                g1_v[pl.ds(l2 * L, L)] = g1c2
                ng2 = jnp.min(g1c2)
                g2_v[...] = jnp.where(iot == l2, ng2, g2v)

            return 0

        lax.fori_loop(0, m_rep, rep_body, 0)

        stat_v[...] = jnp.where(iot == 0, k_count,
                                jnp.where(iot == 1, nvalid, 0))
        pltpu.sync_copy(pri_v, pri_out)
        pltpu.sync_copy(cand_v, cand_out)
        pltpu.sync_copy(stat_v, stats_out)


@functools.partial(
    pl.kernel,
    out_type=jax.ShapeDtypeStruct((POOL_SIZE, SUMMARY_DIM), jnp.float32),
    mesh=_mesh,
    scratch_types=[
        pltpu.VMEM((ROWS_PER_TILE,), jnp.int32),
        pltpu.VMEM((ROWS_PER_TILE, SUMMARY_DIM), jnp.float32),
        pltpu.VMEM((ROWS_PER_TILE, SUMMARY_DIM), jnp.float32),
        pltpu.VMEM((L,), jnp.int32),
        pltpu.SemaphoreType.DMA,
    ],
)
def _k2(summaries_hbm, pool_hbm, cand_hbm, stats_hbm, pool_out,
        idx_v, rows_v, pol_v, stat_v, sem):
    wid = lax.axis_index("s") * 2 + lax.axis_index("c")
    base = wid * ROWS_PER_TILE
    iot = _iota()

    pltpu.sync_copy(cand_hbm.at[pl.ds(base, ROWS_PER_TILE)], idx_v)
    pltpu.sync_copy(stats_hbm, stat_v)
    stat = stat_v[...]
    k_count = jnp.max(jnp.where(iot == 0, stat, jnp.int32(-2147483648)))

    @pl.when(base + ROWS_PER_TILE <= k_count)
    def _():
        pltpu.async_copy(summaries_hbm.at[idx_v], rows_v, sem).wait()
        pltpu.sync_copy(rows_v, pool_out.at[pl.ds(base, ROWS_PER_TILE)])

    @pl.when(base >= k_count)
    def _():
        pltpu.sync_copy(pool_hbm.at[pl.ds(base, ROWS_PER_TILE)], rows_v)
        pltpu.sync_copy(rows_v, pool_out.at[pl.ds(base, ROWS_PER_TILE)])

    @pl.when(jnp.logical_and(base < k_count, base + ROWS_PER_TILE > k_count))
    def _():
        def clamp_body(i, _):
            c = idx_v[pl.ds(i * L, L)]
            idx_v[pl.ds(i * L, L)] = jnp.maximum(c, 0)
            return 0

        lax.fori_loop(0, ROWS_PER_TILE // L, clamp_body, 0)
        pltpu.async_copy(summaries_hbm.at[idx_v], rows_v, sem).wait()
        pltpu.sync_copy(pool_hbm.at[pl.ds(base, ROWS_PER_TILE)], pol_v)

        def fix_body(r, _):
            @pl.when(base + r >= k_count)
            def _():
                def col_body(c8, _):
                    rows_v[r, pl.ds(c8 * L, L)] = pol_v[r, pl.ds(c8 * L, L)]
                    return 0

                lax.fori_loop(0, SUMMARY_DIM // L, col_body, 0)

            return 0

        lax.fori_loop(0, ROWS_PER_TILE, fix_body, 0)
        pltpu.sync_copy(rows_v, pool_out.at[pl.ds(base, ROWS_PER_TILE)])


def kernel(summaries, scores, pool, priorities):
    pri_out, cand, stats = _k1(scores, priorities)
    pool_out = _k2(summaries, pool, cand, stats)
    return pool_out, pri_out, stats[0]


# speedup vs baseline: 1052.4920x; 1052.4920x over previous
import functools

import jax
import jax.numpy as jnp
from jax import lax
from jax.experimental import pallas as pl
from jax.experimental.pallas import tpu as pltpu
from jax.experimental.pallas import tpu_sc as plsc

N_CAND = 16384
POOL_SIZE = 4096
SUMMARY_DIM = 128
THRESHOLD = 0.5
L = 16
NW = 32
ROWS_PER_TILE = POOL_SIZE // NW

_mesh = plsc.VectorSubcoreMesh(core_axis_name="c", subcore_axis_name="s")


def _iota():
    return lax.iota(jnp.int32, L)


@functools.partial(
    pl.kernel,
    out_type=[
        jax.ShapeDtypeStruct((POOL_SIZE,), jnp.float32),
        jax.ShapeDtypeStruct((POOL_SIZE,), jnp.int32),
        jax.ShapeDtypeStruct((L,), jnp.int32),
    ],
    mesh=_mesh,
    scratch_types=[
        pltpu.VMEM((N_CAND,), jnp.float32),
        pltpu.VMEM((N_CAND,), jnp.float32),
        pltpu.VMEM((N_CAND,), jnp.int32),
        pltpu.VMEM((POOL_SIZE,), jnp.float32),
        pltpu.VMEM((POOL_SIZE,), jnp.int32),
        pltpu.VMEM((POOL_SIZE // L,), jnp.float32),
        pltpu.VMEM((L,), jnp.float32),
        pltpu.VMEM((L,), jnp.int32),
    ],
    compiler_params=pltpu.CompilerParams(needs_layout_passes=False),
)
def _k1(scores_hbm, priorities_hbm, pri_out, cand_out, stats_out,
        sc_v, vsc_v, vid_v, pri_v, cand_v, g1_v, g2_v, stat_v):
    wid = lax.axis_index("s") * 2 + lax.axis_index("c")

    @pl.when(wid == 0)
    def _():
        pltpu.sync_copy(scores_hbm, sc_v)
        pltpu.sync_copy(priorities_hbm, pri_v)
        iot = _iota()

        def compact_body(c, off_v):
            v = sc_v[pl.ds(c * L, L)]
            msk = v > THRESHOLD
            incl = jnp.cumsum(jnp.where(msk, 1, 0))
            pos = off_v + incl - 1
            ids = c * L + iot
            plsc.store_scatter(vsc_v, [pos], v, mask=msk)
            plsc.store_scatter(vid_v, [pos], ids, mask=msk)
            return off_v + plsc.all_reduce_population_count(msk)

        off_v = lax.fori_loop(0, N_CAND // L, compact_body,
                              jnp.zeros((L,), jnp.int32))
        nvalid = jnp.max(off_v)
        k_count = jnp.minimum(nvalid, POOL_SIZE)
        m_rep = nvalid - k_count

        def insert_body(ci, _):
            base = ci * L
            msk = (base + iot) < k_count
            scv = vsc_v[pl.ds(base, L)]
            idv = vid_v[pl.ds(base, L)]
            pold = pri_v[pl.ds(base, L)]
            pri_v[pl.ds(base, L)] = jnp.where(msk, scv, pold)
            cand_v[pl.ds(base, L)] = jnp.where(msk, idv, -1)
            return 0

        lax.fori_loop(0, POOL_SIZE // L, insert_body, 0)

        lane0 = iot == 0

        def g1_body(j, _):
            x = pri_v[pl.ds(j * L, L)]
            mn = jnp.min(x)
            plsc.store_scatter(g1_v, [jnp.full((L,), j, jnp.int32)],
                               jnp.full((L,), mn, jnp.float32), mask=lane0)
            return 0

        lax.fori_loop(0, POOL_SIZE // L, g1_body, 0)

        def g2_body(g, _):
            x = g1_v[pl.ds(g * L, L)]
            mn = jnp.min(x)
            plsc.store_scatter(g2_v, [jnp.full((L,), g, jnp.int32)],
                               jnp.full((L,), mn, jnp.float32), mask=lane0)
            return 0

        lax.fori_loop(0, L, g2_body, 0)

        def rep_body(t, _):
            posv = jnp.full((L,), POOL_SIZE + t, jnp.int32)
            rv = plsc.load_gather(vsc_v, [posv])
            g2v = g2_v[...]

            @pl.when(jnp.any(rv > g2v))
            def _():
                gmin = jnp.min(g2v)
                l2 = jnp.min(jnp.where(g2v == gmin, iot, L))
                g1c = g1_v[pl.ds(l2 * L, L)]
                l1 = jnp.min(jnp.where(g1c == gmin, iot, L))
                grp = l2 * L + l1
                pc = pri_v[pl.ds(grp * L, L)]
                l0 = jnp.min(jnp.where(pc == gmin, iot, L))
                newp = jnp.where(iot == l0, rv, pc)
                pri_v[pl.ds(grp * L, L)] = newp
                idv = plsc.load_gather(vid_v, [posv])
                cc = cand_v[pl.ds(grp * L, L)]
                cand_v[pl.ds(grp * L, L)] = jnp.where(iot == l0, idv, cc)
                ng1 = jnp.min(newp)
                g1c2 = jnp.where(iot == l1, ng1, g1c)
                g1_v[pl.ds(l2 * L, L)] = g1c2
                ng2 = jnp.min(g1c2)
                g2_v[...] = jnp.where(iot == l2, ng2, g2v)

            return 0

        lax.fori_loop(0, m_rep, rep_body, 0)

        stat_v[...] = jnp.where(iot == 0, k_count,
                                jnp.where(iot == 1, nvalid, 0))
        pltpu.sync_copy(pri_v, pri_out)
        pltpu.sync_copy(cand_v, cand_out)
        pltpu.sync_copy(stat_v, stats_out)


@functools.partial(
    pl.kernel,
    out_type=jax.ShapeDtypeStruct((POOL_SIZE, SUMMARY_DIM), jnp.float32),
    mesh=_mesh,
    scratch_types=[
        pltpu.VMEM((ROWS_PER_TILE,), jnp.int32),
        pltpu.VMEM((ROWS_PER_TILE, SUMMARY_DIM), jnp.float32),
        pltpu.VMEM((ROWS_PER_TILE, SUMMARY_DIM), jnp.float32),
        pltpu.VMEM((L,), jnp.int32),
        pltpu.SemaphoreType.DMA,
    ],
    compiler_params=pltpu.CompilerParams(needs_layout_passes=False),
)
def _k2(summaries_hbm, pool_hbm, cand_hbm, stats_hbm, pool_out,
        idx_v, rows_v, pol_v, stat_v, sem):
    wid = lax.axis_index("s") * 2 + lax.axis_index("c")
    base = wid * ROWS_PER_TILE
    iot = _iota()

    pltpu.sync_copy(cand_hbm.at[pl.ds(base, ROWS_PER_TILE)], idx_v)
    pltpu.sync_copy(stats_hbm, stat_v)
    stat = stat_v[...]
    k_count = jnp.max(jnp.where(iot == 0, stat, jnp.int32(-2147483648)))

    @pl.when(base + ROWS_PER_TILE <= k_count)
    def _():
        pltpu.async_copy(summaries_hbm.at[idx_v], rows_v, sem).wait()
        pltpu.sync_copy(rows_v, pool_out.at[pl.ds(base, ROWS_PER_TILE)])

    @pl.when(base >= k_count)
    def _():
        pltpu.sync_copy(pool_hbm.at[pl.ds(base, ROWS_PER_TILE)], rows_v)
        pltpu.sync_copy(rows_v, pool_out.at[pl.ds(base, ROWS_PER_TILE)])

    @pl.when(jnp.logical_and(base < k_count, base + ROWS_PER_TILE > k_count))
    def _():
        def clamp_body(i, _):
            c = idx_v[pl.ds(i * L, L)]
            idx_v[pl.ds(i * L, L)] = jnp.maximum(c, 0)
            return 0

        lax.fori_loop(0, ROWS_PER_TILE // L, clamp_body, 0)
        pltpu.async_copy(summaries_hbm.at[idx_v], rows_v, sem).wait()
        pltpu.sync_copy(pool_hbm.at[pl.ds(base, ROWS_PER_TILE)], pol_v)

        def fix_body(r, _):
            @pl.when(base + r >= k_count)
            def _():
                def col_body(c8, _):
                    rows_v[r, pl.ds(c8 * L, L)] = pol_v[r, pl.ds(c8 * L, L)]
                    return 0

                lax.fori_loop(0, SUMMARY_DIM // L, col_body, 0)

            return 0

        lax.fori_loop(0, ROWS_PER_TILE, fix_body, 0)
        pltpu.sync_copy(rows_v, pool_out.at[pl.ds(base, ROWS_PER_TILE)])


def kernel(summaries, scores, pool, priorities):
    pri_out, cand, stats = _k1(scores, priorities)
    pool_out = _k2(summaries, pool, cand, stats)
    return pool_out, pri_out, stats[0]


# speedup vs baseline: 1078.2211x; 1.0244x over previous
import functools

import jax
import jax.numpy as jnp
from jax import lax
from jax.experimental import pallas as pl
from jax.experimental.pallas import tpu as pltpu
from jax.experimental.pallas import tpu_sc as plsc

N_CAND = 16384
POOL_SIZE = 4096
SUMMARY_DIM = 128
THRESHOLD = 0.5
L = 16
NW = 32
ROWS_PER_TILE = POOL_SIZE // NW

_mesh = plsc.VectorSubcoreMesh(core_axis_name="c", subcore_axis_name="s")


def _iota():
    return lax.iota(jnp.int32, L)


@functools.partial(
    pl.kernel,
    out_type=[
        jax.ShapeDtypeStruct((POOL_SIZE,), jnp.float32),
        jax.ShapeDtypeStruct((POOL_SIZE,), jnp.int32),
        jax.ShapeDtypeStruct((L,), jnp.int32),
    ],
    mesh=_mesh,
    scratch_types=[
        pltpu.VMEM((N_CAND,), jnp.float32),
        pltpu.VMEM((N_CAND,), jnp.float32),
        pltpu.VMEM((N_CAND,), jnp.int32),
        pltpu.VMEM((POOL_SIZE,), jnp.float32),
        pltpu.VMEM((POOL_SIZE,), jnp.int32),
        pltpu.VMEM((POOL_SIZE // L,), jnp.float32),
        pltpu.VMEM((L,), jnp.int32),
    ],
    compiler_params=pltpu.CompilerParams(needs_layout_passes=False),
)
def _k1(scores_hbm, priorities_hbm, pri_out, cand_out, stats_out,
        sc_v, vsc_v, vid_v, pri_v, cand_v, g1_v, stat_v):
    wid = lax.axis_index("s") * 2 + lax.axis_index("c")

    @pl.when(wid == 0)
    def _():
        pltpu.sync_copy(scores_hbm, sc_v)
        pltpu.sync_copy(priorities_hbm, pri_v)
        iot = _iota()

        def compact_body(c, off_v):
            v = sc_v[pl.ds(c * L, L)]
            msk = v > THRESHOLD
            incl = jnp.cumsum(jnp.where(msk, 1, 0))
            pos = off_v + incl - 1
            ids = c * L + iot
            plsc.store_scatter(vsc_v, [pos], v, mask=msk)
            plsc.store_scatter(vid_v, [pos], ids, mask=msk)
            return off_v + plsc.all_reduce_population_count(msk)

        off_v = lax.fori_loop(0, N_CAND // L, compact_body,
                              jnp.zeros((L,), jnp.int32))
        nvalid = jnp.max(off_v)
        k_count = jnp.minimum(nvalid, POOL_SIZE)
        m_rep = nvalid - k_count

        lane0 = iot == 0

        def insert_body(ci, _):
            base = ci * L
            msk = (base + iot) < k_count
            scv = vsc_v[pl.ds(base, L)]
            idv = vid_v[pl.ds(base, L)]
            pold = pri_v[pl.ds(base, L)]
            pnew = jnp.where(msk, scv, pold)
            pri_v[pl.ds(base, L)] = pnew
            cand_v[pl.ds(base, L)] = jnp.where(msk, idv, -1)
            mn = jnp.min(pnew)
            plsc.store_scatter(g1_v, [jnp.full((L,), ci, jnp.int32)],
                               jnp.full((L,), mn, jnp.float32), mask=lane0)
            return 0

        lax.fori_loop(0, POOL_SIZE // L, insert_body, 0)

        def g2_body(g, acc):
            x = g1_v[pl.ds(g * L, L)]
            return jnp.where(iot == g, jnp.min(x), acc)

        g2v0 = lax.fori_loop(0, L, g2_body, jnp.zeros((L,), jnp.float32))

        def rep_body(t, g2v):
            posv = jnp.full((L,), POOL_SIZE + t, jnp.int32)
            rv = plsc.load_gather(vsc_v, [posv])

            def accept(g2v):
                gmin = jnp.min(g2v)
                l2 = jnp.min(jnp.where(g2v == gmin, iot, L))
                g1c = g1_v[pl.ds(l2 * L, L)]
                l1 = jnp.min(jnp.where(g1c == gmin, iot, L))
                grp = l2 * L + l1
                pc = pri_v[pl.ds(grp * L, L)]
                l0 = jnp.min(jnp.where(pc == gmin, iot, L))
                newp = jnp.where(iot == l0, rv, pc)
                pri_v[pl.ds(grp * L, L)] = newp
                idv = plsc.load_gather(vid_v, [posv])
                cc = cand_v[pl.ds(grp * L, L)]
                cand_v[pl.ds(grp * L, L)] = jnp.where(iot == l0, idv, cc)
                ng1 = jnp.min(newp)
                g1c2 = jnp.where(iot == l1, ng1, g1c)
                g1_v[pl.ds(l2 * L, L)] = g1c2
                ng2 = jnp.min(g1c2)
                return jnp.where(iot == l2, ng2, g2v)

            return lax.cond(jnp.any(rv > g2v), accept, lambda g: g, g2v)

        lax.fori_loop(0, m_rep, rep_body, g2v0)

        stat_v[...] = jnp.where(iot == 0, k_count,
                                jnp.where(iot == 1, nvalid, 0))
        pltpu.sync_copy(pri_v, pri_out)
        pltpu.sync_copy(cand_v, cand_out)
        pltpu.sync_copy(stat_v, stats_out)


@functools.partial(
    pl.kernel,
    out_type=jax.ShapeDtypeStruct((POOL_SIZE, SUMMARY_DIM), jnp.float32),
    mesh=_mesh,
    scratch_types=[
        pltpu.VMEM((ROWS_PER_TILE,), jnp.int32),
        pltpu.VMEM((ROWS_PER_TILE, SUMMARY_DIM), jnp.float32),
        pltpu.VMEM((ROWS_PER_TILE, SUMMARY_DIM), jnp.float32),
        pltpu.VMEM((L,), jnp.int32),
        pltpu.SemaphoreType.DMA,
    ],
    compiler_params=pltpu.CompilerParams(needs_layout_passes=False),
)
def _k2(summaries_hbm, pool_hbm, cand_hbm, stats_hbm, pool_out,
        idx_v, rows_v, pol_v, stat_v, sem):
    wid = lax.axis_index("s") * 2 + lax.axis_index("c")
    base = wid * ROWS_PER_TILE
    iot = _iota()

    pltpu.sync_copy(cand_hbm.at[pl.ds(base, ROWS_PER_TILE)], idx_v)
    pltpu.sync_copy(stats_hbm, stat_v)
    stat = stat_v[...]
    k_count = jnp.max(jnp.where(iot == 0, stat, jnp.int32(-2147483648)))

    @pl.when(base + ROWS_PER_TILE <= k_count)
    def _():
        pltpu.async_copy(summaries_hbm.at[idx_v], rows_v, sem).wait()
        pltpu.sync_copy(rows_v, pool_out.at[pl.ds(base, ROWS_PER_TILE)])

    @pl.when(base >= k_count)
    def _():
        pltpu.sync_copy(pool_hbm.at[pl.ds(base, ROWS_PER_TILE)], rows_v)
        pltpu.sync_copy(rows_v, pool_out.at[pl.ds(base, ROWS_PER_TILE)])

    @pl.when(jnp.logical_and(base < k_count, base + ROWS_PER_TILE > k_count))
    def _():
        def clamp_body(i, _):
            c = idx_v[pl.ds(i * L, L)]
            idx_v[pl.ds(i * L, L)] = jnp.maximum(c, 0)
            return 0

        lax.fori_loop(0, ROWS_PER_TILE // L, clamp_body, 0)
        pltpu.async_copy(summaries_hbm.at[idx_v], rows_v, sem).wait()
        pltpu.sync_copy(pool_hbm.at[pl.ds(base, ROWS_PER_TILE)], pol_v)

        def fix_body(r, _):
            @pl.when(base + r >= k_count)
            def _():
                def col_body(c8, _):
                    rows_v[r, pl.ds(c8 * L, L)] = pol_v[r, pl.ds(c8 * L, L)]
                    return 0

                lax.fori_loop(0, SUMMARY_DIM // L, col_body, 0)

            return 0

        lax.fori_loop(0, ROWS_PER_TILE, fix_body, 0)
        pltpu.sync_copy(rows_v, pool_out.at[pl.ds(base, ROWS_PER_TILE)])


def kernel(summaries, scores, pool, priorities):
    pri_out, cand, stats = _k1(scores, priorities)
    pool_out = _k2(summaries, pool, cand, stats)
    return pool_out, pri_out, stats[0]


# speedup vs baseline: 1222.2532x; 1.1336x over previous
import functools

import jax
import jax.numpy as jnp
from jax import lax
from jax.experimental import pallas as pl
from jax.experimental.pallas import tpu as pltpu
from jax.experimental.pallas import tpu_sc as plsc

N_CAND = 16384
POOL_SIZE = 4096
SUMMARY_DIM = 128
THRESHOLD = 0.5
L = 16
NW = 32
ROWS_PER_TILE = POOL_SIZE // NW

_mesh = plsc.VectorSubcoreMesh(core_axis_name="c", subcore_axis_name="s")


def _iota():
    return lax.iota(jnp.int32, L)


@functools.partial(
    pl.kernel,
    out_type=[
        jax.ShapeDtypeStruct((POOL_SIZE,), jnp.float32),
        jax.ShapeDtypeStruct((POOL_SIZE,), jnp.int32),
        jax.ShapeDtypeStruct((L,), jnp.int32),
    ],
    mesh=_mesh,
    scratch_types=[
        pltpu.VMEM((N_CAND,), jnp.float32),
        pltpu.VMEM((N_CAND,), jnp.float32),
        pltpu.VMEM((N_CAND,), jnp.int32),
        pltpu.VMEM((POOL_SIZE,), jnp.float32),
        pltpu.VMEM((POOL_SIZE,), jnp.int32),
        pltpu.VMEM((POOL_SIZE // L,), jnp.float32),
        pltpu.VMEM((L,), jnp.int32),
    ],
    compiler_params=pltpu.CompilerParams(needs_layout_passes=False),
)
def _k1(scores_hbm, priorities_hbm, pri_out, cand_out, stats_out,
        sc_v, vsc_v, vid_v, pri_v, cand_v, g1_v, stat_v):
    wid = lax.axis_index("s") * 2 + lax.axis_index("c")

    @pl.when(wid == 0)
    def _():
        pltpu.sync_copy(scores_hbm, sc_v)
        pltpu.sync_copy(priorities_hbm, pri_v)
        iot = _iota()

        def compact_body(c, off_v):
            v = sc_v[pl.ds(c * L, L)]
            msk = v > THRESHOLD
            incl = jnp.cumsum(jnp.where(msk, 1, 0))
            pos = off_v + incl - 1
            ids = c * L + iot
            plsc.store_scatter(vsc_v, [pos], v, mask=msk)
            plsc.store_scatter(vid_v, [pos], ids, mask=msk)
            return off_v + plsc.all_reduce_population_count(msk)

        off_v = lax.fori_loop(0, N_CAND // L, compact_body,
                              jnp.zeros((L,), jnp.int32))
        nvalid = jnp.max(off_v)
        k_count = jnp.minimum(nvalid, POOL_SIZE)
        m_rep = nvalid - k_count

        lane0 = iot == 0

        def insert_body(ci, _):
            base = ci * L
            msk = (base + iot) < k_count
            scv = vsc_v[pl.ds(base, L)]
            idv = vid_v[pl.ds(base, L)]
            pold = pri_v[pl.ds(base, L)]
            pnew = jnp.where(msk, scv, pold)
            pri_v[pl.ds(base, L)] = pnew
            cand_v[pl.ds(base, L)] = jnp.where(msk, idv, -1)
            mn = jnp.min(pnew)
            plsc.store_scatter(g1_v, [jnp.full((L,), ci, jnp.int32)],
                               jnp.full((L,), mn, jnp.float32), mask=lane0)
            return 0

        lax.fori_loop(0, POOL_SIZE // L, insert_body, 0)

        def g2_body(g, acc):
            x = g1_v[pl.ds(g * L, L)]
            return jnp.where(iot == g, jnp.min(x), acc)

        g2v0 = lax.fori_loop(0, L, g2_body, jnp.zeros((L,), jnp.float32))

        def rep_body(t, g2v):
            posv = jnp.full((L,), POOL_SIZE + t, jnp.int32)
            rv = plsc.load_gather(vsc_v, [posv])
            acc = jnp.any(rv > g2v)
            gmin = jnp.min(g2v)
            l2 = jnp.min(jnp.where(g2v == gmin, iot, L))
            g1c = g1_v[pl.ds(l2 * L, L)]
            l1 = jnp.min(jnp.where(g1c == gmin, iot, L))
            grp = l2 * L + l1
            pc = pri_v[pl.ds(grp * L, L)]
            l0 = jnp.min(jnp.where(pc == gmin, iot, L))
            newp = jnp.where(jnp.logical_and(iot == l0, acc), rv, pc)
            pri_v[pl.ds(grp * L, L)] = newp
            idv = plsc.load_gather(vid_v, [posv])
            cc = cand_v[pl.ds(grp * L, L)]
            cand_v[pl.ds(grp * L, L)] = jnp.where(
                jnp.logical_and(iot == l0, acc), idv, cc)
            ng1 = jnp.min(newp)
            g1c2 = jnp.where(jnp.logical_and(iot == l1, acc), ng1, g1c)
            g1_v[pl.ds(l2 * L, L)] = g1c2
            ng2 = jnp.min(g1c2)
            return jnp.where(jnp.logical_and(iot == l2, acc), ng2, g2v)

        lax.fori_loop(0, m_rep, rep_body, g2v0)

        stat_v[...] = jnp.where(iot == 0, k_count,
                                jnp.where(iot == 1, nvalid, 0))
        pltpu.sync_copy(pri_v, pri_out)
        pltpu.sync_copy(cand_v, cand_out)
        pltpu.sync_copy(stat_v, stats_out)


@functools.partial(
    pl.kernel,
    out_type=jax.ShapeDtypeStruct((POOL_SIZE, SUMMARY_DIM), jnp.float32),
    mesh=_mesh,
    scratch_types=[
        pltpu.VMEM((ROWS_PER_TILE,), jnp.int32),
        pltpu.VMEM((ROWS_PER_TILE, SUMMARY_DIM), jnp.float32),
        pltpu.VMEM((ROWS_PER_TILE, SUMMARY_DIM), jnp.float32),
        pltpu.VMEM((L,), jnp.int32),
        pltpu.SemaphoreType.DMA,
    ],
    compiler_params=pltpu.CompilerParams(needs_layout_passes=False),
)
def _k2(summaries_hbm, pool_hbm, cand_hbm, stats_hbm, pool_out,
        idx_v, rows_v, pol_v, stat_v, sem):
    wid = lax.axis_index("s") * 2 + lax.axis_index("c")
    base = wid * ROWS_PER_TILE
    iot = _iota()

    pltpu.sync_copy(cand_hbm.at[pl.ds(base, ROWS_PER_TILE)], idx_v)
    pltpu.sync_copy(stats_hbm, stat_v)
    stat = stat_v[...]
    k_count = jnp.max(jnp.where(iot == 0, stat, jnp.int32(-2147483648)))

    @pl.when(base + ROWS_PER_TILE <= k_count)
    def _():
        pltpu.async_copy(summaries_hbm.at[idx_v], rows_v, sem).wait()
        pltpu.sync_copy(rows_v, pool_out.at[pl.ds(base, ROWS_PER_TILE)])

    @pl.when(base >= k_count)
    def _():
        pltpu.sync_copy(pool_hbm.at[pl.ds(base, ROWS_PER_TILE)], rows_v)
        pltpu.sync_copy(rows_v, pool_out.at[pl.ds(base, ROWS_PER_TILE)])

    @pl.when(jnp.logical_and(base < k_count, base + ROWS_PER_TILE > k_count))
    def _():
        def clamp_body(i, _):
            c = idx_v[pl.ds(i * L, L)]
            idx_v[pl.ds(i * L, L)] = jnp.maximum(c, 0)
            return 0

        lax.fori_loop(0, ROWS_PER_TILE // L, clamp_body, 0)
        pltpu.async_copy(summaries_hbm.at[idx_v], rows_v, sem).wait()
        pltpu.sync_copy(pool_hbm.at[pl.ds(base, ROWS_PER_TILE)], pol_v)

        def fix_body(r, _):
            @pl.when(base + r >= k_count)
            def _():
                def col_body(c8, _):
                    rows_v[r, pl.ds(c8 * L, L)] = pol_v[r, pl.ds(c8 * L, L)]
                    return 0

                lax.fori_loop(0, SUMMARY_DIM // L, col_body, 0)

            return 0

        lax.fori_loop(0, ROWS_PER_TILE, fix_body, 0)
        pltpu.sync_copy(rows_v, pool_out.at[pl.ds(base, ROWS_PER_TILE)])


def kernel(summaries, scores, pool, priorities):
    pri_out, cand, stats = _k1(scores, priorities)
    pool_out = _k2(summaries, pool, cand, stats)
    return pool_out, pri_out, stats[0]


# speedup vs baseline: 1227.6158x; 1.0044x over previous
import functools

import jax
import jax.numpy as jnp
from jax import lax
from jax.experimental import pallas as pl
from jax.experimental.pallas import tpu as pltpu
from jax.experimental.pallas import tpu_sc as plsc

N_CAND = 16384
POOL_SIZE = 4096
SUMMARY_DIM = 128
THRESHOLD = 0.5
L = 16
NW = 32
ROWS_PER_TILE = POOL_SIZE // NW

_mesh = plsc.VectorSubcoreMesh(core_axis_name="c", subcore_axis_name="s")


def _iota():
    return lax.iota(jnp.int32, L)


@functools.partial(
    pl.kernel,
    out_type=[
        jax.ShapeDtypeStruct((POOL_SIZE,), jnp.float32),
        jax.ShapeDtypeStruct((POOL_SIZE,), jnp.int32),
        jax.ShapeDtypeStruct((L,), jnp.int32),
    ],
    mesh=_mesh,
    scratch_types=[
        pltpu.VMEM((N_CAND,), jnp.float32),
        pltpu.VMEM((N_CAND,), jnp.float32),
        pltpu.VMEM((N_CAND,), jnp.int32),
        pltpu.VMEM((POOL_SIZE,), jnp.float32),
        pltpu.VMEM((POOL_SIZE,), jnp.int32),
        pltpu.VMEM((POOL_SIZE // L,), jnp.float32),
        pltpu.VMEM((L,), jnp.int32),
    ],
    compiler_params=pltpu.CompilerParams(needs_layout_passes=False),
)
def _k1(scores_hbm, priorities_hbm, pri_out, cand_out, stats_out,
        sc_v, vsc_v, vid_v, pri_v, cand_v, g1_v, stat_v):
    wid = lax.axis_index("s") * 2 + lax.axis_index("c")

    @pl.when(wid == 0)
    def _():
        pltpu.sync_copy(scores_hbm, sc_v)
        pltpu.sync_copy(priorities_hbm, pri_v)
        iot = _iota()

        def compact_body(c, off_v):
            v = sc_v[pl.ds(c * L, L)]
            msk = v > THRESHOLD
            incl = jnp.cumsum(jnp.where(msk, 1, 0))
            pos = off_v + incl - 1
            ids = c * L + iot
            plsc.store_scatter(vsc_v, [pos], v, mask=msk)
            plsc.store_scatter(vid_v, [pos], ids, mask=msk)
            return off_v + plsc.all_reduce_population_count(msk)

        off_v = lax.fori_loop(0, N_CAND // L, compact_body,
                              jnp.zeros((L,), jnp.int32))
        nvalid = jnp.max(off_v)
        k_count = jnp.minimum(nvalid, POOL_SIZE)
        m_rep = nvalid - k_count

        lane0 = iot == 0

        def insert_body(ci, _):
            base = ci * L
            msk = (base + iot) < k_count
            scv = vsc_v[pl.ds(base, L)]
            idv = vid_v[pl.ds(base, L)]
            pold = pri_v[pl.ds(base, L)]
            pnew = jnp.where(msk, scv, pold)
            pri_v[pl.ds(base, L)] = pnew
            cand_v[pl.ds(base, L)] = jnp.where(msk, idv, -1)
            mn = jnp.min(pnew)
            plsc.store_scatter(g1_v, [jnp.full((L,), ci, jnp.int32)],
                               jnp.full((L,), mn, jnp.float32), mask=lane0)
            return 0

        lax.fori_loop(0, POOL_SIZE // L, insert_body, 0)

        def g2_body(g, acc):
            x = g1_v[pl.ds(g * L, L)]
            return jnp.where(iot == g, jnp.min(x), acc)

        g2v0 = lax.fori_loop(0, L, g2_body, jnp.zeros((L,), jnp.float32))

        def rep_body(t, carry):
            g2v, rv, idv = carry
            pos_n = jnp.minimum(POOL_SIZE + t + 1, N_CAND - 1)
            posv_n = jnp.full((L,), pos_n, jnp.int32)
            rv_n = plsc.load_gather(vsc_v, [posv_n])
            idv_n = plsc.load_gather(vid_v, [posv_n])
            acc = jnp.any(rv > g2v)
            gmin = jnp.min(g2v)
            l2 = jnp.min(jnp.where(g2v == gmin, iot, L))
            g1c = g1_v[pl.ds(l2 * L, L)]
            l1 = jnp.min(jnp.where(g1c == gmin, iot, L))
            grp = l2 * L + l1
            pc = pri_v[pl.ds(grp * L, L)]
            l0 = jnp.min(jnp.where(pc == gmin, iot, L))
            newp = jnp.where(jnp.logical_and(iot == l0, acc), rv, pc)
            pri_v[pl.ds(grp * L, L)] = newp
            cc = cand_v[pl.ds(grp * L, L)]
            cand_v[pl.ds(grp * L, L)] = jnp.where(
                jnp.logical_and(iot == l0, acc), idv, cc)
            ng1 = jnp.min(newp)
            g1c2 = jnp.where(jnp.logical_and(iot == l1, acc), ng1, g1c)
            g1_v[pl.ds(l2 * L, L)] = g1c2
            ng2 = jnp.min(g1c2)
            g2v = jnp.where(jnp.logical_and(iot == l2, acc), ng2, g2v)
            return (g2v, rv_n, idv_n)

        posv0 = jnp.full((L,), POOL_SIZE, jnp.int32)
        rv0 = plsc.load_gather(vsc_v, [posv0])
        idv0 = plsc.load_gather(vid_v, [posv0])
        lax.fori_loop(0, m_rep, rep_body, (g2v0, rv0, idv0))

        stat_v[...] = jnp.where(iot == 0, k_count,
                                jnp.where(iot == 1, nvalid, 0))
        pltpu.sync_copy(pri_v, pri_out)
        pltpu.sync_copy(cand_v, cand_out)
        pltpu.sync_copy(stat_v, stats_out)


@functools.partial(
    pl.kernel,
    out_type=jax.ShapeDtypeStruct((POOL_SIZE, SUMMARY_DIM), jnp.float32),
    mesh=_mesh,
    scratch_types=[
        pltpu.VMEM((ROWS_PER_TILE,), jnp.int32),
        pltpu.VMEM((ROWS_PER_TILE, SUMMARY_DIM), jnp.float32),
        pltpu.VMEM((ROWS_PER_TILE, SUMMARY_DIM), jnp.float32),
        pltpu.VMEM((L,), jnp.int32),
        pltpu.SemaphoreType.DMA,
    ],
    compiler_params=pltpu.CompilerParams(needs_layout_passes=False),
)
def _k2(summaries_hbm, pool_hbm, cand_hbm, stats_hbm, pool_out,
        idx_v, rows_v, pol_v, stat_v, sem):
    wid = lax.axis_index("s") * 2 + lax.axis_index("c")
    base = wid * ROWS_PER_TILE
    iot = _iota()

    pltpu.sync_copy(cand_hbm.at[pl.ds(base, ROWS_PER_TILE)], idx_v)
    pltpu.sync_copy(stats_hbm, stat_v)
    stat = stat_v[...]
    k_count = jnp.max(jnp.where(iot == 0, stat, jnp.int32(-2147483648)))

    @pl.when(base + ROWS_PER_TILE <= k_count)
    def _():
        pltpu.async_copy(summaries_hbm.at[idx_v], rows_v, sem).wait()
        pltpu.sync_copy(rows_v, pool_out.at[pl.ds(base, ROWS_PER_TILE)])

    @pl.when(base >= k_count)
    def _():
        pltpu.sync_copy(pool_hbm.at[pl.ds(base, ROWS_PER_TILE)], rows_v)
        pltpu.sync_copy(rows_v, pool_out.at[pl.ds(base, ROWS_PER_TILE)])

    @pl.when(jnp.logical_and(base < k_count, base + ROWS_PER_TILE > k_count))
    def _():
        def clamp_body(i, _):
            c = idx_v[pl.ds(i * L, L)]
            idx_v[pl.ds(i * L, L)] = jnp.maximum(c, 0)
            return 0

        lax.fori_loop(0, ROWS_PER_TILE // L, clamp_body, 0)
        pltpu.async_copy(summaries_hbm.at[idx_v], rows_v, sem).wait()
        pltpu.sync_copy(pool_hbm.at[pl.ds(base, ROWS_PER_TILE)], pol_v)

        def fix_body(r, _):
            @pl.when(base + r >= k_count)
            def _():
                def col_body(c8, _):
                    rows_v[r, pl.ds(c8 * L, L)] = pol_v[r, pl.ds(c8 * L, L)]
                    return 0

                lax.fori_loop(0, SUMMARY_DIM // L, col_body, 0)

            return 0

        lax.fori_loop(0, ROWS_PER_TILE, fix_body, 0)
        pltpu.sync_copy(rows_v, pool_out.at[pl.ds(base, ROWS_PER_TILE)])


def kernel(summaries, scores, pool, priorities):
    pri_out, cand, stats = _k1(scores, priorities)
    pool_out = _k2(summaries, pool, cand, stats)
    return pool_out, pri_out, stats[0]
